# baseline (device time: 103880 ns/iter reference)
import jax
import jax.numpy as jnp
from jax import lax
from jax.experimental import pallas as pl
from jax.experimental.pallas import tpu as pltpu

N_DEV = 8
B_LOC = 2
SQ = 128
SKV = 128
HQ = 32
H_LOC = HQ // N_DEV
DH = 64
D_MODEL = 512
SCALE = 0.125


def kernel(x, Wq, K_ext, V_ext, Wo):
    me = lax.axis_index("i")

    x2d = x.reshape(B_LOC * SQ, D_MODEL)
    K_loc = lax.dynamic_slice_in_dim(K_ext, me * B_LOC, B_LOC, axis=0)
    V_loc = lax.dynamic_slice_in_dim(V_ext, me * B_LOC, B_LOC, axis=0)
    K_r = K_loc.transpose(2, 0, 1, 3).reshape(HQ * B_LOC, SKV, DH)
    V_r = V_loc.transpose(2, 0, 1, 3).reshape(HQ * B_LOC, SKV, DH)

    blk = H_LOC * DH

    def body(x_ref, wq_ref, k_ref, v_ref, wo_ref, out_ref,
             wq_comm, wo_comm, ctx_ref,
             wq_send, wq_recv, wo_send, wo_recv):
        my = lax.axis_index("i")
        left = lax.rem(my + N_DEV - 1, N_DEV)
        right = lax.rem(my + 1, N_DEV)

        barrier_sem = pltpu.get_barrier_semaphore()
        for nbr in (left, right):
            pl.semaphore_signal(
                barrier_sem, inc=1,
                device_id=(nbr,), device_id_type=pl.DeviceIdType.MESH,
            )
        pl.semaphore_wait(barrier_sem, 2)

        qb = lax.broadcasted_iota(jnp.int32, (SQ, SKV), 0) // 64
        kb = lax.broadcasted_iota(jnp.int32, (SQ, SKV), 1) // 64
        mask = (qb == kb) | (kb == 0) | (lax.rem(qb + kb, 3) == 0)

        def compute_block(origin, wq_blk, wo_blk, is_first):
            q_all = jnp.dot(x_ref[:, :], wq_blk,
                            preferred_element_type=jnp.float32)
            for b in range(B_LOC):
                for hl in range(H_LOC):
                    head = (origin * H_LOC + hl) * B_LOC + b
                    q = q_all[b * SQ:(b + 1) * SQ, hl * DH:(hl + 1) * DH]
                    k = k_ref[head]
                    s = lax.dot_general(
                        q, k, (((1,), (1,)), ((), ())),
                        preferred_element_type=jnp.float32) * SCALE
                    s = jnp.where(mask, s, -1e9)
                    m = jnp.max(s, axis=1, keepdims=True)
                    w = jnp.exp(s - m)
                    w = w / jnp.sum(w, axis=1, keepdims=True)
                    c = jnp.dot(w, v_ref[head],
                                preferred_element_type=jnp.float32)
                    ctx_ref[b * SQ:(b + 1) * SQ, hl * DH:(hl + 1) * DH] = c
            contrib = jnp.dot(ctx_ref[:, :], wo_blk,
                              preferred_element_type=jnp.float32)
            if is_first:
                out_ref[:, :] = contrib
            else:
                out_ref[:, :] += contrib

        for h in range(N_DEV - 1):
            src_wq = wq_ref if h == 0 else wq_comm.at[h - 1]
            src_wo = wo_ref if h == 0 else wo_comm.at[h - 1]
            rd_wq = pltpu.make_async_remote_copy(
                src_ref=src_wq, dst_ref=wq_comm.at[h],
                send_sem=wq_send.at[h], recv_sem=wq_recv.at[h],
                device_id=(right,), device_id_type=pl.DeviceIdType.MESH,
            )
            rd_wo = pltpu.make_async_remote_copy(
                src_ref=src_wo, dst_ref=wo_comm.at[h],
                send_sem=wo_send.at[h], recv_sem=wo_recv.at[h],
                device_id=(right,), device_id_type=pl.DeviceIdType.MESH,
            )
            rd_wq.start()
            rd_wo.start()

            origin = lax.rem(my + N_DEV - h, N_DEV)
            if h == 0:
                compute_block(origin, wq_ref[:, :], wo_ref[:, :], True)
            else:
                compute_block(origin, wq_comm[h - 1], wo_comm[h - 1], False)

            rd_wq.wait()
            rd_wo.wait()

        origin = lax.rem(my + 1, N_DEV)
        compute_block(origin, wq_comm[N_DEV - 2], wo_comm[N_DEV - 2], False)

    out2d = pl.pallas_call(
        body,
        out_shape=jax.ShapeDtypeStruct((B_LOC * SQ, D_MODEL), jnp.float32),
        in_specs=[pl.BlockSpec(memory_space=pltpu.VMEM)] * 5,
        out_specs=pl.BlockSpec(memory_space=pltpu.VMEM),
        scratch_shapes=[
            pltpu.VMEM((N_DEV - 1, D_MODEL, blk), jnp.float32),
            pltpu.VMEM((N_DEV - 1, blk, D_MODEL), jnp.float32),
            pltpu.VMEM((B_LOC * SQ, blk), jnp.float32),
            pltpu.SemaphoreType.DMA((N_DEV - 1,)),
            pltpu.SemaphoreType.DMA((N_DEV - 1,)),
            pltpu.SemaphoreType.DMA((N_DEV - 1,)),
            pltpu.SemaphoreType.DMA((N_DEV - 1,)),
        ],
        compiler_params=pltpu.CompilerParams(collective_id=0),
    )(x2d, Wq, K_r, V_r, Wo)

    return out2d.reshape(B_LOC, SQ, D_MODEL)


# device time: 67622 ns/iter; 1.5362x vs baseline; 1.5362x over previous
import jax
import jax.numpy as jnp
from jax import lax
from jax.experimental import pallas as pl
from jax.experimental.pallas import tpu as pltpu

N_DEV = 8
B_LOC = 2
SQ = 128
SKV = 128
HQ = 32
H_LOC = HQ // N_DEV
DH = 64
D_MODEL = 512
SCALE = 0.125


def kernel(x, Wq, K_ext, V_ext, Wo):
    me = lax.axis_index("i")

    x2d = x.reshape(B_LOC * SQ, D_MODEL)
    K_loc = lax.dynamic_slice_in_dim(K_ext, me * B_LOC, B_LOC, axis=0)
    V_loc = lax.dynamic_slice_in_dim(V_ext, me * B_LOC, B_LOC, axis=0)
    K_r = K_loc.transpose(2, 0, 1, 3).reshape(HQ * B_LOC, SKV, DH)
    V_r = V_loc.transpose(2, 0, 1, 3).reshape(HQ * B_LOC, SKV, DH)

    blk = H_LOC * DH

    def body(x_ref, wq_ref, k_ref, v_ref, wo_ref, out_ref,
             wq_comm, wo_comm, ctx_all,
             wq_send, wq_recv, wo_send, wo_recv):
        my = lax.axis_index("i")
        left = lax.rem(my + N_DEV - 1, N_DEV)
        right = lax.rem(my + 1, N_DEV)

        barrier_sem = pltpu.get_barrier_semaphore()
        for nbr in (left, right):
            pl.semaphore_signal(
                barrier_sem, inc=1,
                device_id=(nbr,), device_id_type=pl.DeviceIdType.MESH,
            )
        pl.semaphore_wait(barrier_sem, 2)

        qb = lax.broadcasted_iota(jnp.int32, (SQ, SKV), 0) // 64
        kb = lax.broadcasted_iota(jnp.int32, (SQ, SKV), 1) // 64
        mask = (qb == kb) | (kb == 0) | (lax.rem(qb + kb, 3) == 0)

        def attention(origin, wq_blk, ctx_slot):
            q_all = jnp.dot(x_ref[:, :], wq_blk,
                            preferred_element_type=jnp.float32)
            for b in range(B_LOC):
                for hl in range(H_LOC):
                    head = (origin * H_LOC + hl) * B_LOC + b
                    q = q_all[b * SQ:(b + 1) * SQ, hl * DH:(hl + 1) * DH]
                    k = k_ref[head]
                    s = lax.dot_general(
                        q, k, (((1,), (1,)), ((), ())),
                        preferred_element_type=jnp.float32) * SCALE
                    s = jnp.where(mask, s, -1e9)
                    m = jnp.max(s, axis=1, keepdims=True)
                    w = jnp.exp(s - m)
                    w = w / jnp.sum(w, axis=1, keepdims=True)
                    c = jnp.dot(w, v_ref[head],
                                preferred_element_type=jnp.float32)
                    ctx_all[ctx_slot, b * SQ:(b + 1) * SQ,
                            hl * DH:(hl + 1) * DH] = c

        def contrib(d, is_first=False):
            ctx_blk = ctx_all[(N_DEV - d) % N_DEV]
            wo_blk = wo_ref[:, :] if d == 0 else wo_comm[d - 1]
            c = jnp.dot(ctx_blk, wo_blk, preferred_element_type=jnp.float32)
            if is_first:
                out_ref[:, :] = c
            else:
                out_ref[:, :] += c

        contrib_at = {4: [4], 5: [3, 5], 6: [2, 6]}

        for h in range(N_DEV - 1):
            src_wq = wq_ref if h == 0 else wq_comm.at[h - 1]
            src_wo = wo_ref if h == 0 else wo_comm.at[h - 1]
            rd_wq = pltpu.make_async_remote_copy(
                src_ref=src_wq, dst_ref=wq_comm.at[h],
                send_sem=wq_send.at[h], recv_sem=wq_recv.at[h],
                device_id=(right,), device_id_type=pl.DeviceIdType.MESH,
            )
            rd_wo = pltpu.make_async_remote_copy(
                src_ref=src_wo, dst_ref=wo_comm.at[h],
                send_sem=wo_send.at[h], recv_sem=wo_recv.at[h],
                device_id=(left,), device_id_type=pl.DeviceIdType.MESH,
            )
            rd_wq.start()
            rd_wo.start()

            origin = lax.rem(my + N_DEV - h, N_DEV)
            attention(origin, wq_ref[:, :] if h == 0 else wq_comm[h - 1], h)
            if h == 0:
                contrib(0, is_first=True)
            for d in contrib_at.get(h, ()):
                contrib(d)

            rd_wq.wait()
            rd_wo.wait()

        attention(lax.rem(my + 1, N_DEV), wq_comm[N_DEV - 2], N_DEV - 1)
        contrib(1)
        contrib(7)

    out2d = pl.pallas_call(
        body,
        out_shape=jax.ShapeDtypeStruct((B_LOC * SQ, D_MODEL), jnp.float32),
        in_specs=[pl.BlockSpec(memory_space=pltpu.VMEM)] * 5,
        out_specs=pl.BlockSpec(memory_space=pltpu.VMEM),
        scratch_shapes=[
            pltpu.VMEM((N_DEV - 1, D_MODEL, blk), jnp.float32),
            pltpu.VMEM((N_DEV - 1, blk, D_MODEL), jnp.float32),
            pltpu.VMEM((N_DEV, B_LOC * SQ, blk), jnp.float32),
            pltpu.SemaphoreType.DMA((N_DEV - 1,)),
            pltpu.SemaphoreType.DMA((N_DEV - 1,)),
            pltpu.SemaphoreType.DMA((N_DEV - 1,)),
            pltpu.SemaphoreType.DMA((N_DEV - 1,)),
        ],
        compiler_params=pltpu.CompilerParams(collective_id=0),
    )(x2d, Wq, K_r, V_r, Wo)

    return out2d.reshape(B_LOC, SQ, D_MODEL)


# device time: 50556 ns/iter; 2.0548x vs baseline; 1.3376x over previous
import jax
import jax.numpy as jnp
from jax import lax
from jax.experimental import pallas as pl
from jax.experimental.pallas import tpu as pltpu

N_DEV = 8
B_LOC = 2
SQ = 128
SKV = 128
HQ = 32
H_LOC = HQ // N_DEV
DH = 64
D_MODEL = 512
SCALE = 0.125


def kernel(x, Wq, K_ext, V_ext, Wo):
    me = lax.axis_index("i")

    x2d = x.reshape(B_LOC * SQ, D_MODEL).astype(jnp.bfloat16)
    Wq = Wq.astype(jnp.bfloat16)
    Wo = Wo.astype(jnp.bfloat16)
    K_loc = lax.dynamic_slice_in_dim(K_ext, me * B_LOC, B_LOC, axis=0)
    V_loc = lax.dynamic_slice_in_dim(V_ext, me * B_LOC, B_LOC, axis=0)
    K_r = K_loc.transpose(2, 0, 1, 3).reshape(HQ * B_LOC, SKV, DH)
    K_r = K_r.astype(jnp.bfloat16)
    V_r = V_loc.transpose(2, 0, 1, 3).reshape(HQ * B_LOC, SKV, DH)
    V_r = V_r.astype(jnp.bfloat16)

    blk = H_LOC * DH

    def body(x_ref, wq_ref, k_ref, v_ref, wo_ref, out_ref,
             wq_comm, wo_comm, ctx_all,
             wq_send, wq_recv, wo_send, wo_recv):
        my = lax.axis_index("i")
        left = lax.rem(my + N_DEV - 1, N_DEV)
        right = lax.rem(my + 1, N_DEV)

        barrier_sem = pltpu.get_barrier_semaphore()
        for nbr in (left, right):
            pl.semaphore_signal(
                barrier_sem, inc=1,
                device_id=(nbr,), device_id_type=pl.DeviceIdType.MESH,
            )
        pl.semaphore_wait(barrier_sem, 2)

        qb = lax.broadcasted_iota(jnp.int32, (SQ, SKV), 0) // 64
        kb = lax.broadcasted_iota(jnp.int32, (SQ, SKV), 1) // 64
        mask = (qb == kb) | (kb == 0) | (lax.rem(qb + kb, 3) == 0)

        def attention(origin, wq_blk, ctx_slot):
            q_all = jnp.dot(x_ref[:, :], wq_blk,
                            preferred_element_type=jnp.float32)
            q_all = q_all.astype(jnp.bfloat16)
            for b in range(B_LOC):
                for hl in range(H_LOC):
                    head = (origin * H_LOC + hl) * B_LOC + b
                    q = q_all[b * SQ:(b + 1) * SQ, hl * DH:(hl + 1) * DH]
                    k = k_ref[head]
                    s = lax.dot_general(
                        q, k, (((1,), (1,)), ((), ())),
                        preferred_element_type=jnp.float32) * SCALE
                    s = jnp.where(mask, s, -1e9)
                    m = jnp.max(s, axis=1, keepdims=True)
                    w = jnp.exp(s - m)
                    w = (w / jnp.sum(w, axis=1, keepdims=True)
                         ).astype(jnp.bfloat16)
                    c = jnp.dot(w, v_ref[head],
                                preferred_element_type=jnp.float32)
                    ctx_all[ctx_slot, b * SQ:(b + 1) * SQ,
                            hl * DH:(hl + 1) * DH] = c.astype(jnp.bfloat16)

        def contrib(d, is_first=False):
            ctx_blk = ctx_all[(N_DEV - d) % N_DEV]
            wo_blk = wo_ref[:, :] if d == 0 else wo_comm[d - 1]
            c = jnp.dot(ctx_blk, wo_blk, preferred_element_type=jnp.float32)
            if is_first:
                out_ref[:, :] = c
            else:
                out_ref[:, :] += c

        contrib_at = {4: [4], 5: [3, 5], 6: [2, 6]}

        for h in range(N_DEV - 1):
            src_wq = wq_ref if h == 0 else wq_comm.at[h - 1]
            src_wo = wo_ref if h == 0 else wo_comm.at[h - 1]
            rd_wq = pltpu.make_async_remote_copy(
                src_ref=src_wq, dst_ref=wq_comm.at[h],
                send_sem=wq_send.at[h], recv_sem=wq_recv.at[h],
                device_id=(right,), device_id_type=pl.DeviceIdType.MESH,
            )
            rd_wo = pltpu.make_async_remote_copy(
                src_ref=src_wo, dst_ref=wo_comm.at[h],
                send_sem=wo_send.at[h], recv_sem=wo_recv.at[h],
                device_id=(left,), device_id_type=pl.DeviceIdType.MESH,
            )
            rd_wq.start()
            rd_wo.start()

            origin = lax.rem(my + N_DEV - h, N_DEV)
            attention(origin, wq_ref[:, :] if h == 0 else wq_comm[h - 1], h)
            if h == 0:
                contrib(0, is_first=True)
            for d in contrib_at.get(h, ()):
                contrib(d)

            rd_wq.wait()
            rd_wo.wait()

        attention(lax.rem(my + 1, N_DEV), wq_comm[N_DEV - 2], N_DEV - 1)
        contrib(1)
        contrib(7)

    out2d = pl.pallas_call(
        body,
        out_shape=jax.ShapeDtypeStruct((B_LOC * SQ, D_MODEL), jnp.float32),
        in_specs=[pl.BlockSpec(memory_space=pltpu.VMEM)] * 5,
        out_specs=pl.BlockSpec(memory_space=pltpu.VMEM),
        scratch_shapes=[
            pltpu.VMEM((N_DEV - 1, D_MODEL, blk), jnp.bfloat16),
            pltpu.VMEM((N_DEV - 1, blk, D_MODEL), jnp.bfloat16),
            pltpu.VMEM((N_DEV, B_LOC * SQ, blk), jnp.bfloat16),
            pltpu.SemaphoreType.DMA((N_DEV - 1,)),
            pltpu.SemaphoreType.DMA((N_DEV - 1,)),
            pltpu.SemaphoreType.DMA((N_DEV - 1,)),
            pltpu.SemaphoreType.DMA((N_DEV - 1,)),
        ],
        compiler_params=pltpu.CompilerParams(collective_id=0),
    )(x2d, Wq, K_r, V_r, Wo)

    return out2d.reshape(B_LOC, SQ, D_MODEL)


# device time: 45666 ns/iter; 2.2748x vs baseline; 1.1071x over previous
import jax
import jax.numpy as jnp
from jax import lax
from jax.experimental import pallas as pl
from jax.experimental.pallas import tpu as pltpu

N_DEV = 8
B_LOC = 2
SQ = 128
SKV = 128
HQ = 32
H_LOC = HQ // N_DEV
DH = 64
D_MODEL = 512
SCALE = 0.125

BF = jnp.bfloat16


def _p_of_bits(xb, yb):
    return 2 * yb + (xb + yb - 2 * xb * yb)


def _bits_of_pos(o):
    p = o % 4
    xb = 1 if p in (1, 2) else 0
    yb = 1 if p >= 2 else 0
    return xb, yb, o // 4


def kernel(x, Wq, K_ext, V_ext, Wo):
    me = lax.axis_index("i")

    x2d = x.reshape(B_LOC * SQ, D_MODEL)
    K_loc = lax.dynamic_slice_in_dim(K_ext, me * B_LOC, B_LOC, axis=0)
    V_loc = lax.dynamic_slice_in_dim(V_ext, me * B_LOC, B_LOC, axis=0)
    K_r = K_loc.astype(BF).transpose(2, 0, 1, 3).reshape(HQ * B_LOC, SKV, DH)
    V_r = V_loc.astype(BF).transpose(2, 0, 1, 3).reshape(HQ * B_LOC, SKV, DH)

    blk = H_LOC * DH

    def body(x_ref, wq_ref, k_ref, v_ref, wo_ref, out_ref,
             x_bf, wq_all, wo_all, ctx_all,
             wq_send, wq_recv, wo_send, wo_recv):
        my = lax.axis_index("i")

        p4 = lax.rem(my, 4)
        zb = my // 4
        z4 = my - p4
        xb = jnp.where((p4 == 1) | (p4 == 2), 1, 0)
        yb = jnp.where(p4 >= 2, 1, 0)
        partner_x = z4 + (p4 + 1 - 2 * lax.rem(p4, 2))
        partner_y = z4 + (3 - p4)
        partner_z = lax.rem(my + 4, N_DEV)

        s_wq_me = zb * 4 + yb * 2 + xb
        s_wo_me = xb * 4 + zb * 2 + yb

        def wq_slot_origin(s):
            xb_ = lax.rem(s, 2)
            yb_ = lax.rem(s // 2, 2)
            return (s // 4) * 4 + _p_of_bits(xb_, yb_)

        barrier_sem = pltpu.get_barrier_semaphore()
        for nbr in (partner_x, partner_y, partner_z):
            pl.semaphore_signal(
                barrier_sem, inc=1,
                device_id=(nbr,), device_id_type=pl.DeviceIdType.MESH,
            )
        pl.semaphore_wait(barrier_sem, 3)

        x_bf[:, :] = x_ref[:, :].astype(BF)
        wq_all[s_wq_me, :, :] = wq_ref[:, :].astype(BF)
        wo_all[s_wo_me, :, :] = wo_ref[:, :].astype(BF)

        qb_i = lax.broadcasted_iota(jnp.int32, (SQ, SKV), 0) // 64
        kb_i = lax.broadcasted_iota(jnp.int32, (SQ, SKV), 1) // 64
        mask = (qb_i == kb_i) | (kb_i == 0) | (lax.rem(qb_i + kb_i, 3) == 0)

        def attention(origin, slot):
            q_all = jnp.dot(x_bf[:, :], wq_all[slot],
                            preferred_element_type=jnp.float32)
            q_all = q_all.astype(BF)
            for b in range(B_LOC):
                for hl in range(H_LOC):
                    head = (origin * H_LOC + hl) * B_LOC + b
                    q = q_all[b * SQ:(b + 1) * SQ, hl * DH:(hl + 1) * DH]
                    k = k_ref[head]
                    s = lax.dot_general(
                        q, k, (((1,), (1,)), ((), ())),
                        preferred_element_type=jnp.float32) * SCALE
                    s = jnp.where(mask, s, -1e9)
                    m = jnp.max(s, axis=1, keepdims=True)
                    w = jnp.exp(s - m)
                    w = (w / jnp.sum(w, axis=1, keepdims=True)).astype(BF)
                    c = jnp.dot(w, v_ref[head],
                                preferred_element_type=jnp.float32)
                    ctx_all[slot, b * SQ:(b + 1) * SQ,
                            hl * DH:(hl + 1) * DH] = c.astype(BF)

        pending_sends = []

        def send(buf_all, slot, n, sem_arr, sem_idx, recv_sem, target):
            rd = pltpu.make_async_remote_copy(
                src_ref=buf_all.at[pl.ds(slot, n)],
                dst_ref=buf_all.at[pl.ds(slot, n)],
                send_sem=sem_arr.at[sem_idx],
                recv_sem=recv_sem,
                device_id=(target,), device_id_type=pl.DeviceIdType.MESH,
            )
            rd.start()
            pending_sends.append(rd)

        def wait_recv(buf_all, slot, n, recv_sem):
            rd = pltpu.make_async_remote_copy(
                src_ref=buf_all.at[pl.ds(slot, n)],
                dst_ref=buf_all.at[pl.ds(slot, n)],
                send_sem=wq_send.at[0],
                recv_sem=recv_sem,
                device_id=(my,), device_id_type=pl.DeviceIdType.MESH,
            )
            rd.wait_recv()

        iq0 = s_wq_me + 1 - 2 * xb
        io0 = s_wo_me + 1 - 2 * yb
        send(wq_all, s_wq_me, 1, wq_send, 0, wq_recv.at[s_wq_me], partner_x)
        send(wo_all, s_wo_me, 1, wo_send, 0, wo_recv.at[0], partner_y)
        attention(my, s_wq_me)
        wait_recv(wq_all, iq0, 1, wq_recv.at[iq0])
        attention(partner_x, iq0)

        qb1 = s_wq_me - xb
        ob1 = s_wo_me - yb
        qin1 = z4 + (1 - yb) * 2
        wait_recv(wo_all, io0, 1, wo_recv.at[0])
        send(wq_all, qb1, 1, wq_send, 1, wq_recv.at[qb1], partner_y)
        send(wq_all, qb1 + 1, 1, wq_send, 2, wq_recv.at[qb1 + 1], partner_y)
        send(wo_all, ob1, 2, wo_send, 1, wo_recv.at[1], partner_z)
        for j in range(2):
            wait_recv(wq_all, qin1 + j, 1, wq_recv.at[qin1 + j])
            attention(wq_slot_origin(qin1 + j), qin1 + j)

        qq2 = zb * 4
        oq2 = xb * 4
        qin2 = (1 - zb) * 4
        wait_recv(wo_all, xb * 4 + (1 - zb) * 2, 2, wo_recv.at[1])
        for j in range(4):
            send(wq_all, qq2 + j, 1, wq_send, 3 + j,
                 wq_recv.at[qq2 + j], partner_z)
        send(wo_all, oq2, 4, wo_send, 2, wo_recv.at[2], partner_x)
        for j in range(4):
            wait_recv(wq_all, qin2 + j, 1, wq_recv.at[qin2 + j])
            attention(wq_slot_origin(qin2 + j), qin2 + j)
        wait_recv(wo_all, (1 - xb) * 4, 4, wo_recv.at[2])

        for s in range(N_DEV):
            sx, sy, sz = s & 1, (s >> 1) & 1, s >> 2
            mslot = sx * 4 + sz * 2 + sy
            c = jnp.dot(ctx_all[s], wo_all[mslot],
                        preferred_element_type=jnp.float32)
            if s == 0:
                out_ref[:, :] = c
            else:
                out_ref[:, :] += c

        for rd in pending_sends:
            rd.wait_send()

    out2d = pl.pallas_call(
        body,
        out_shape=jax.ShapeDtypeStruct((B_LOC * SQ, D_MODEL), jnp.float32),
        in_specs=[pl.BlockSpec(memory_space=pltpu.VMEM)] * 5,
        out_specs=pl.BlockSpec(memory_space=pltpu.VMEM),
        scratch_shapes=[
            pltpu.VMEM((B_LOC * SQ, D_MODEL), BF),
            pltpu.VMEM((N_DEV, D_MODEL, blk), BF),
            pltpu.VMEM((N_DEV, blk, D_MODEL), BF),
            pltpu.VMEM((N_DEV, B_LOC * SQ, blk), BF),
            pltpu.SemaphoreType.DMA((7,)),
            pltpu.SemaphoreType.DMA((N_DEV,)),
            pltpu.SemaphoreType.DMA((3,)),
            pltpu.SemaphoreType.DMA((3,)),
        ],
        compiler_params=pltpu.CompilerParams(collective_id=0),
    )(x2d, Wq, K_r, V_r, Wo)

    return out2d.reshape(B_LOC, SQ, D_MODEL)


# device time: 40752 ns/iter; 2.5491x vs baseline; 1.1206x over previous
import jax
import jax.numpy as jnp
from jax import lax
from jax.experimental import pallas as pl
from jax.experimental.pallas import tpu as pltpu

N_DEV = 8
B_LOC = 2
SQ = 128
SKV = 128
HQ = 32
H_LOC = HQ // N_DEV
DH = 64
D_MODEL = 512
SCALE = 0.125

BF = jnp.bfloat16


def _p_of_bits(xb, yb):
    return 2 * yb + (xb + yb - 2 * xb * yb)


def _bits_of_pos(o):
    p = o % 4
    xb = 1 if p in (1, 2) else 0
    yb = 1 if p >= 2 else 0
    return xb, yb, o // 4


def kernel(x, Wq, K_ext, V_ext, Wo):
    me = lax.axis_index("i")

    x2d = x.reshape(B_LOC * SQ, D_MODEL)
    K_loc = lax.dynamic_slice_in_dim(K_ext, me * B_LOC, B_LOC, axis=0)
    V_loc = lax.dynamic_slice_in_dim(V_ext, me * B_LOC, B_LOC, axis=0)
    K_r = K_loc.astype(BF).transpose(2, 0, 1, 3).reshape(HQ * B_LOC, SKV, DH)
    V_r = V_loc.astype(BF).transpose(2, 0, 1, 3).reshape(HQ * B_LOC, SKV, DH)

    blk = H_LOC * DH

    def body(x_ref, wq_ref, k_ref, v_ref, wo_ref, out_ref,
             x_bf, wq_all, wo_all, ctx_all,
             wq_send, wq_recv, wo_send, wo_recv):
        my = lax.axis_index("i")

        p4 = lax.rem(my, 4)
        zb = my // 4
        z4 = my - p4
        xb = jnp.where((p4 == 1) | (p4 == 2), 1, 0)
        yb = jnp.where(p4 >= 2, 1, 0)
        partner_x = z4 + (p4 + 1 - 2 * lax.rem(p4, 2))
        partner_y = z4 + (3 - p4)
        partner_z = lax.rem(my + 4, N_DEV)

        s_wq_me = zb * 4 + yb * 2 + xb
        s_wo_me = xb * 4 + zb * 2 + yb

        def wq_slot_origin(s):
            xb_ = lax.rem(s, 2)
            yb_ = lax.rem(s // 2, 2)
            return (s // 4) * 4 + _p_of_bits(xb_, yb_)

        barrier_sem = pltpu.get_barrier_semaphore()
        for nbr in (partner_x, partner_y, partner_z):
            pl.semaphore_signal(
                barrier_sem, inc=1,
                device_id=(nbr,), device_id_type=pl.DeviceIdType.MESH,
            )
        pl.semaphore_wait(barrier_sem, 3)

        x_bf[:, :] = x_ref[:, :].astype(BF)
        wq_all[s_wq_me, :, :] = wq_ref[:, :].astype(BF)
        wo_all[s_wo_me, :, :] = wo_ref[:, :].astype(BF)

        qb_i = lax.broadcasted_iota(jnp.int32, (SQ, SKV), 0) // 64
        kb_i = lax.broadcasted_iota(jnp.int32, (SQ, SKV), 1) // 64
        mask = (qb_i == kb_i) | (kb_i == 0) | (lax.rem(qb_i + kb_i, 3) == 0)

        def attention(origin, slot):
            q_all = jnp.dot(x_bf[:, :], wq_all[slot],
                            preferred_element_type=jnp.float32)
            q_all = q_all.astype(BF)
            for b in range(B_LOC):
                for hl in range(H_LOC):
                    head = (origin * H_LOC + hl) * B_LOC + b
                    q = q_all[b * SQ:(b + 1) * SQ, hl * DH:(hl + 1) * DH]
                    k = k_ref[head]
                    s = lax.dot_general(
                        q, k, (((1,), (1,)), ((), ())),
                        preferred_element_type=jnp.float32) * SCALE
                    s = jnp.where(mask, s, -1e9)
                    m = jnp.max(s, axis=1, keepdims=True)
                    w = jnp.exp(s - m)
                    w = (w / jnp.sum(w, axis=1, keepdims=True)).astype(BF)
                    c = jnp.dot(w, v_ref[head],
                                preferred_element_type=jnp.float32)
                    ctx_all[slot, b * SQ:(b + 1) * SQ,
                            hl * DH:(hl + 1) * DH] = c.astype(BF)

        pending_sends = []

        def send(buf_all, slot, n, sem_arr, sem_idx, recv_sem, target):
            rd = pltpu.make_async_remote_copy(
                src_ref=buf_all.at[pl.ds(slot, n)],
                dst_ref=buf_all.at[pl.ds(slot, n)],
                send_sem=sem_arr.at[sem_idx],
                recv_sem=recv_sem,
                device_id=(target,), device_id_type=pl.DeviceIdType.MESH,
            )
            rd.start()
            pending_sends.append(rd)

        def wait_recv(buf_all, slot, n, recv_sem):
            rd = pltpu.make_async_remote_copy(
                src_ref=buf_all.at[pl.ds(slot, n)],
                dst_ref=buf_all.at[pl.ds(slot, n)],
                send_sem=wq_send.at[0],
                recv_sem=recv_sem,
                device_id=(my,), device_id_type=pl.DeviceIdType.MESH,
            )
            rd.wait_recv()


        iq0 = s_wq_me + 1 - 2 * xb
        io0 = s_wo_me + 1 - 2 * yb
        send(wq_all, s_wq_me, 1, wq_send, 0, wq_recv.at[s_wq_me], partner_x)
        send(wo_all, s_wo_me, 1, wo_send, 0, wo_recv.at[0], partner_y)
        attention(my, s_wq_me)
        wait_recv(wq_all, iq0, 1, wq_recv.at[iq0])
        wait_recv(wo_all, io0, 1, wo_recv.at[0])

        qb1 = s_wq_me - xb
        ob1 = s_wo_me - yb
        qin1 = z4 + (1 - yb) * 2
        send(wq_all, qb1, 1, wq_send, 1, wq_recv.at[qb1], partner_y)
        send(wq_all, qb1 + 1, 1, wq_send, 2, wq_recv.at[qb1 + 1], partner_y)
        send(wo_all, ob1, 2, wo_send, 1, wo_recv.at[1], partner_z)
        attention(partner_x, iq0)
        wait_recv(wq_all, qin1, 1, wq_recv.at[qin1])
        wait_recv(wq_all, qin1 + 1, 1, wq_recv.at[qin1 + 1])
        wait_recv(wo_all, xb * 4 + (1 - zb) * 2, 2, wo_recv.at[1])

        qq2 = zb * 4
        oq2 = xb * 4
        qin2 = (1 - zb) * 4
        for j in range(4):
            send(wq_all, qq2 + j, 1, wq_send, 3 + j,
                 wq_recv.at[qq2 + j], partner_z)
        send(wo_all, oq2, 4, wo_send, 2, wo_recv.at[2], partner_x)
        attention(wq_slot_origin(qin1), qin1)
        attention(wq_slot_origin(qin1 + 1), qin1 + 1)
        for j in range(4):
            wait_recv(wq_all, qin2 + j, 1, wq_recv.at[qin2 + j])
            attention(wq_slot_origin(qin2 + j), qin2 + j)
        wait_recv(wo_all, (1 - xb) * 4, 4, wo_recv.at[2])

        for s in range(N_DEV):
            sx, sy, sz = s & 1, (s >> 1) & 1, s >> 2
            mslot = sx * 4 + sz * 2 + sy
            c = jnp.dot(ctx_all[s], wo_all[mslot],
                        preferred_element_type=jnp.float32)
            if s == 0:
                out_ref[:, :] = c
            else:
                out_ref[:, :] += c

        for rd in pending_sends:
            rd.wait_send()

    out2d = pl.pallas_call(
        body,
        out_shape=jax.ShapeDtypeStruct((B_LOC * SQ, D_MODEL), jnp.float32),
        in_specs=[pl.BlockSpec(memory_space=pltpu.VMEM)] * 5,
        out_specs=pl.BlockSpec(memory_space=pltpu.VMEM),
        scratch_shapes=[
            pltpu.VMEM((B_LOC * SQ, D_MODEL), BF),
            pltpu.VMEM((N_DEV, D_MODEL, blk), BF),
            pltpu.VMEM((N_DEV, blk, D_MODEL), BF),
            pltpu.VMEM((N_DEV, B_LOC * SQ, blk), BF),
            pltpu.SemaphoreType.DMA((7,)),
            pltpu.SemaphoreType.DMA((N_DEV,)),
            pltpu.SemaphoreType.DMA((3,)),
            pltpu.SemaphoreType.DMA((3,)),
        ],
        compiler_params=pltpu.CompilerParams(collective_id=0),
    )(x2d, Wq, K_r, V_r, Wo)

    return out2d.reshape(B_LOC, SQ, D_MODEL)


# device time: 40565 ns/iter; 2.5608x vs baseline; 1.0046x over previous
import jax
import jax.numpy as jnp
from jax import lax
from jax.experimental import pallas as pl
from jax.experimental.pallas import tpu as pltpu

N_DEV = 8
B_LOC = 2
SQ = 128
SKV = 128
HQ = 32
H_LOC = HQ // N_DEV
DH = 64
D_MODEL = 512
SCALE = 0.125

BF = jnp.bfloat16


def _p_of_bits(xb, yb):
    return 2 * yb + (xb + yb - 2 * xb * yb)


def _bits_of_pos(o):
    p = o % 4
    xb = 1 if p in (1, 2) else 0
    yb = 1 if p >= 2 else 0
    return xb, yb, o // 4


def kernel(x, Wq, K_ext, V_ext, Wo):
    me = lax.axis_index("i")

    x2d = x.reshape(B_LOC * SQ, D_MODEL)
    K_loc = lax.dynamic_slice_in_dim(K_ext, me * B_LOC, B_LOC, axis=0)
    V_loc = lax.dynamic_slice_in_dim(V_ext, me * B_LOC, B_LOC, axis=0)
    K_r = K_loc.astype(BF).transpose(2, 0, 1, 3).reshape(HQ * B_LOC, SKV, DH)
    V_r = V_loc.astype(BF).transpose(2, 0, 1, 3).reshape(HQ * B_LOC, SKV, DH)

    blk = H_LOC * DH

    def body(x_ref, wq_ref, k_ref, v_ref, wo_ref, out_ref,
             x_bf, wq_all, wo_all, ctx_all,
             wq_send, wq_recv, wo_send, wo_recv):
        my = lax.axis_index("i")

        p4 = lax.rem(my, 4)
        zb = my // 4
        z4 = my - p4
        xb = jnp.where((p4 == 1) | (p4 == 2), 1, 0)
        yb = jnp.where(p4 >= 2, 1, 0)
        partner_x = z4 + (p4 + 1 - 2 * lax.rem(p4, 2))
        partner_y = z4 + (3 - p4)
        partner_z = lax.rem(my + 4, N_DEV)

        s_wq_me = zb * 4 + yb * 2 + xb
        s_wo_me = xb * 4 + zb * 2 + yb

        def wq_slot_origin(s):
            xb_ = lax.rem(s, 2)
            yb_ = lax.rem(s // 2, 2)
            return (s // 4) * 4 + _p_of_bits(xb_, yb_)

        barrier_sem = pltpu.get_barrier_semaphore()
        for nbr in (partner_x, partner_y, partner_z):
            pl.semaphore_signal(
                barrier_sem, inc=1,
                device_id=(nbr,), device_id_type=pl.DeviceIdType.MESH,
            )
        pl.semaphore_wait(barrier_sem, 3)

        x_bf[:, :] = x_ref[:, :].astype(BF)
        wq_all[s_wq_me, :, :] = wq_ref[:, :].astype(BF)
        wo_all[s_wo_me, :, :] = wo_ref[:, :].astype(BF)

        qb_i = lax.broadcasted_iota(jnp.int32, (SQ, SKV), 0) // 64
        kb_i = lax.broadcasted_iota(jnp.int32, (SQ, SKV), 1) // 64
        mask = (qb_i == kb_i) | (kb_i == 0) | (lax.rem(qb_i + kb_i, 3) == 0)

        def attention(origin, slot):
            q_all = jnp.dot(x_bf[:, :], wq_all[slot],
                            preferred_element_type=jnp.float32)
            q_all = q_all.astype(BF)
            for b in range(B_LOC):
                for hl in range(H_LOC):
                    head = (origin * H_LOC + hl) * B_LOC + b
                    q = q_all[b * SQ:(b + 1) * SQ, hl * DH:(hl + 1) * DH]
                    k = k_ref[head]
                    s = lax.dot_general(
                        q, k, (((1,), (1,)), ((), ())),
                        preferred_element_type=jnp.float32) * SCALE
                    s = jnp.where(mask, s, -1e9)
                    m = jnp.max(s, axis=1, keepdims=True)
                    w = jnp.exp(s - m)
                    w = (w / jnp.sum(w, axis=1, keepdims=True)).astype(BF)
                    c = jnp.dot(w, v_ref[head],
                                preferred_element_type=jnp.float32)
                    ctx_all[slot, b * SQ:(b + 1) * SQ,
                            hl * DH:(hl + 1) * DH] = c.astype(BF)

        def wo_slot_of(s):
            xb_ = lax.rem(s, 2)
            yb_ = lax.rem(s // 2, 2)
            return xb_ * 4 + (s // 4) * 2 + yb_

        def contrib(wq_slot, is_first=False):
            c = jnp.dot(ctx_all[wq_slot], wo_all[wo_slot_of(wq_slot)],
                        preferred_element_type=jnp.float32)
            if is_first:
                out_ref[:, :] = c
            else:
                out_ref[:, :] += c

        pending_sends = []

        def send(buf_all, slot, n, sem_arr, sem_idx, recv_sem, target):
            rd = pltpu.make_async_remote_copy(
                src_ref=buf_all.at[pl.ds(slot, n)],
                dst_ref=buf_all.at[pl.ds(slot, n)],
                send_sem=sem_arr.at[sem_idx],
                recv_sem=recv_sem,
                device_id=(target,), device_id_type=pl.DeviceIdType.MESH,
            )
            rd.start()
            pending_sends.append(rd)

        def wait_recv(buf_all, slot, n, recv_sem):
            rd = pltpu.make_async_remote_copy(
                src_ref=buf_all.at[pl.ds(slot, n)],
                dst_ref=buf_all.at[pl.ds(slot, n)],
                send_sem=wq_send.at[0],
                recv_sem=recv_sem,
                device_id=(my,), device_id_type=pl.DeviceIdType.MESH,
            )
            rd.wait_recv()


        iq0 = s_wq_me + 1 - 2 * xb
        io0 = s_wo_me + 1 - 2 * yb
        send(wq_all, s_wq_me, 1, wq_send, 0, wq_recv.at[s_wq_me], partner_x)
        send(wo_all, s_wo_me, 1, wo_send, 0, wo_recv.at[0], partner_y)
        attention(my, s_wq_me)
        contrib(s_wq_me, is_first=True)
        wait_recv(wq_all, iq0, 1, wq_recv.at[iq0])
        wait_recv(wo_all, io0, 1, wo_recv.at[0])

        qb1 = s_wq_me - xb
        ob1 = s_wo_me - yb
        qin1 = z4 + (1 - yb) * 2
        send(wq_all, qb1, 1, wq_send, 1, wq_recv.at[qb1], partner_y)
        send(wq_all, qb1 + 1, 1, wq_send, 2, wq_recv.at[qb1 + 1], partner_y)
        send(wo_all, ob1, 2, wo_send, 1, wo_recv.at[1], partner_z)
        attention(partner_x, iq0)
        wait_recv(wq_all, qin1, 1, wq_recv.at[qin1])
        wait_recv(wq_all, qin1 + 1, 1, wq_recv.at[qin1 + 1])
        wait_recv(wo_all, xb * 4 + (1 - zb) * 2, 2, wo_recv.at[1])

        qq2 = zb * 4
        oq2 = xb * 4
        qin2 = (1 - zb) * 4
        q_order = [xb, 2 + xb, 1 - xb, 3 - xb]
        for i, j in enumerate(q_order):
            send(wq_all, qq2 + j, 1, wq_send, 3 + i,
                 wq_recv.at[qq2 + j], partner_z)
        send(wo_all, oq2 + zb * 2, 2, wo_send, 2, wo_recv.at[2], partner_x)
        send(wo_all, oq2 + (1 - zb) * 2, 2, wo_send, 3, wo_recv.at[3],
             partner_x)
        attention(wq_slot_origin(qin1), qin1)
        attention(wq_slot_origin(qin1 + 1), qin1 + 1)
        contrib(qin1 + xb)
        for i, j in enumerate(q_order):
            wait_recv(wq_all, qin2 + j, 1, wq_recv.at[qin2 + j])
            attention(wq_slot_origin(qin2 + j), qin2 + j)
            if i < 2:
                contrib(qin2 + j)
        wait_recv(wo_all, (1 - xb) * 4 + zb * 2, 2, wo_recv.at[2])
        contrib(zb * 4 + 1 - xb)
        contrib(zb * 4 + 3 - xb)
        wait_recv(wo_all, (1 - xb) * 4 + (1 - zb) * 2, 2, wo_recv.at[3])
        contrib((1 - zb) * 4 + 1 - xb)
        contrib((1 - zb) * 4 + 3 - xb)

        for rd in pending_sends:
            rd.wait_send()

    out2d = pl.pallas_call(
        body,
        out_shape=jax.ShapeDtypeStruct((B_LOC * SQ, D_MODEL), jnp.float32),
        in_specs=[pl.BlockSpec(memory_space=pltpu.VMEM)] * 5,
        out_specs=pl.BlockSpec(memory_space=pltpu.VMEM),
        scratch_shapes=[
            pltpu.VMEM((B_LOC * SQ, D_MODEL), BF),
            pltpu.VMEM((N_DEV, D_MODEL, blk), BF),
            pltpu.VMEM((N_DEV, blk, D_MODEL), BF),
            pltpu.VMEM((N_DEV, B_LOC * SQ, blk), BF),
            pltpu.SemaphoreType.DMA((7,)),
            pltpu.SemaphoreType.DMA((N_DEV,)),
            pltpu.SemaphoreType.DMA((4,)),
            pltpu.SemaphoreType.DMA((4,)),
        ],
        compiler_params=pltpu.CompilerParams(collective_id=0),
    )(x2d, Wq, K_r, V_r, Wo)

    return out2d.reshape(B_LOC, SQ, D_MODEL)
